# Initial kernel scaffold; baseline (speedup 1.0000x reference)
import jax
import jax.numpy as jnp
from jax.experimental import pallas as pl


def kernel(x, c, mod_w, mod_b, qkv_w, qkv_b, q_scale, k_scale, proj_w, proj_b, gate_w, eg_w, eu_w, ed_w, sg_w, su_w, sd_w):
    raise NotImplementedError("write your pallas kernel here")


# speedup vs baseline: 1.3465x; 1.3465x over previous
import jax
import jax.numpy as jnp
import numpy as np
from jax import lax
from jax.experimental import pallas as pl
from jax.experimental.pallas import tpu as pltpu

_B, _L, _D, _H = 1, 2048, 1024, 16
_Dh = _D // _H
_E, _K = 8, 2
_F = 4 * _D
_FS = 2 * _D

_LB = 256
_T = 256
_A = _L * _K
_MAXB = _A // _T + _E
_FC = 1024
_NF = _F // _FC


def _mm_nt(a, b):
    return lax.dot_general(a, b, (((1,), (1,)), ((), ())),
                           preferred_element_type=jnp.float32)


def _mm_nn(a, b):
    return lax.dot_general(a, b, (((1,), (0,)), ((), ())),
                           preferred_element_type=jnp.float32)


def _layernorm(x):
    mu = jnp.mean(x, axis=-1, keepdims=True)
    var = jnp.mean((x - mu) ** 2, axis=-1, keepdims=True)
    return (x - mu) * lax.rsqrt(var + 1e-6)


def _mod_body(c_ref, w_ref, b_ref, o_ref):
    cs = jax.nn.silu(c_ref[...])
    o_ref[...] = _mm_nt(cs, w_ref[...]) + b_ref[...]


def _qkv_body(x_ref, s1_ref, sh1_ref, w_ref, b_ref, qs_ref, ks_ref,
              q_out, k_out, v_out):
    x = x_ref[...]
    xm = (1.0 + s1_ref[...]) * _layernorm(x) + sh1_ref[...]
    qkv = _mm_nt(xm, w_ref[...]) + b_ref[...]
    q = qkv[:, :_D]
    k = qkv[:, _D:2 * _D]
    v = qkv[:, 2 * _D:]
    r = lax.broadcasted_iota(jnp.int32, (_D, _H), 0) // _Dh
    c = lax.broadcasted_iota(jnp.int32, (_D, _H), 1)
    m = (r == c).astype(jnp.float32)

    def rms(t, scale_full):
        ssq = _mm_nn(t * t, m)
        rr = lax.rsqrt(ssq / _Dh + 1e-6)
        rrf = _mm_nt(rr, m)
        return t * rrf * scale_full

    q_out[...] = rms(q, qs_ref[...])
    k_out[...] = rms(k, ks_ref[...])
    v_out[...] = v


def _attn_body(q_ref, k_ref, v_ref, o_ref):
    q = q_ref[0]
    k = k_ref[0]
    v = v_ref[0]
    logits = _mm_nt(q, k) * (1.0 / np.sqrt(_Dh))
    mx = jnp.max(logits, axis=-1, keepdims=True)
    p = jnp.exp(logits - mx)
    o = _mm_nn(p, v) / jnp.sum(p, axis=-1, keepdims=True)
    o_ref[0] = o


def _proj_body(o_ref, x_ref, g1_ref, s2_ref, sh2_ref, w_ref, b_ref, gw_ref,
               x1_out, xm2_out, gl_out):
    p = _mm_nt(o_ref[...], w_ref[...]) + b_ref[...]
    x1 = x_ref[...] + g1_ref[...] * p
    xm2 = (1.0 + s2_ref[...]) * _layernorm(x1) + sh2_ref[...]
    x1_out[...] = x1
    xm2_out[...] = xm2
    gl_out[...] = _mm_nt(xm2, gw_ref[...])


def _moe_body(be_ref, bv_ref, xs_ref, eg_ref, eu_ref, ed_ref, out_ref,
              acc_ref):
    f = pl.program_id(0)
    b = pl.program_id(1)
    sl = pl.ds(b * _T, _T)

    @pl.when(f == 0)
    def _():
        acc_ref[sl, :] = jnp.zeros((_T, _D), jnp.float32)

    @pl.when(bv_ref[b] > 0)
    def _():
        xs = xs_ref[...]
        g = _mm_nt(xs, eg_ref[0])
        u = _mm_nt(xs, eu_ref[0])
        h = jax.nn.silu(g) * u
        acc_ref[sl, :] += lax.dot_general(
            h, ed_ref[0], (((1,), (1,)), ((), ())),
            preferred_element_type=jnp.float32)

    out_ref[...] = acc_ref[sl, :]


def _shared_body(xm2_ref, x1_ref, y_ref, g2_ref, sg_ref, su_ref, sd_ref,
                 out_ref):
    t = xm2_ref[...]
    g = _mm_nt(t, sg_ref[...])
    u = _mm_nt(t, su_ref[...])
    h = jax.nn.silu(g) * u
    sh = _mm_nt(h, sd_ref[...])
    out_ref[...] = x1_ref[...] + g2_ref[...] * (y_ref[...] + sh)


def _full(shape):
    return pl.BlockSpec(shape, lambda *_: tuple(0 for _ in shape))


def kernel(x, c, mod_w, mod_b, qkv_w, qkv_b, q_scale, k_scale, proj_w,
           proj_b, gate_w, eg_w, eu_w, ed_w, sg_w, su_w, sd_w):
    f32 = jnp.float32
    xf = x.reshape(_L, _D)
    nl = _L // _LB

    vec = pl.pallas_call(
        _mod_body,
        out_shape=jax.ShapeDtypeStruct((1, 6 * _D), f32),
        in_specs=[_full((1, _D)), _full((6 * _D, _D)), _full((1, 6 * _D))],
        out_specs=_full((1, 6 * _D)),
    )(c.reshape(1, _D), mod_w, mod_b.reshape(1, 6 * _D))
    sh1 = vec[:, 0:_D]
    sc1 = vec[:, _D:2 * _D]
    g1 = vec[:, 2 * _D:3 * _D]
    sh2 = vec[:, 3 * _D:4 * _D]
    sc2 = vec[:, 4 * _D:5 * _D]
    g2 = vec[:, 5 * _D:6 * _D]

    qs_full = jnp.tile(q_scale, _H).reshape(1, _D)
    ks_full = jnp.tile(k_scale, _H).reshape(1, _D)

    row_spec = pl.BlockSpec((_LB, _D), lambda i: (i, 0))
    one_spec = pl.BlockSpec((1, _D), lambda i: (0, 0))
    q, k, v = pl.pallas_call(
        _qkv_body,
        grid=(nl,),
        out_shape=[jax.ShapeDtypeStruct((_L, _D), f32)] * 3,
        in_specs=[row_spec, one_spec, one_spec,
                  pl.BlockSpec((3 * _D, _D), lambda i: (0, 0)),
                  pl.BlockSpec((1, 3 * _D), lambda i: (0, 0)),
                  one_spec, one_spec],
        out_specs=[row_spec] * 3,
    )(xf, sc1, sh1, qkv_w, qkv_b.reshape(1, 3 * _D), qs_full, ks_full)

    qh = q.reshape(_L, _H, _Dh).transpose(1, 0, 2)
    kh = k.reshape(_L, _H, _Dh).transpose(1, 0, 2)
    vh = v.reshape(_L, _H, _Dh).transpose(1, 0, 2)
    head_spec = pl.BlockSpec((1, _L, _Dh), lambda h: (h, 0, 0))
    oh = pl.pallas_call(
        _attn_body,
        grid=(_H,),
        out_shape=jax.ShapeDtypeStruct((_H, _L, _Dh), f32),
        in_specs=[head_spec] * 3,
        out_specs=head_spec,
    )(qh, kh, vh)
    o = oh.transpose(1, 0, 2).reshape(_L, _D)

    gw_pad = jnp.zeros((128, _D), f32).at[:_E].set(gate_w)
    x1, xm2, glog = pl.pallas_call(
        _proj_body,
        grid=(nl,),
        out_shape=[jax.ShapeDtypeStruct((_L, _D), f32),
                   jax.ShapeDtypeStruct((_L, _D), f32),
                   jax.ShapeDtypeStruct((_L, 128), f32)],
        in_specs=[row_spec, row_spec, one_spec, one_spec, one_spec,
                  pl.BlockSpec((_D, _D), lambda i: (0, 0)),
                  one_spec,
                  pl.BlockSpec((128, _D), lambda i: (0, 0))],
        out_specs=[row_spec, row_spec,
                   pl.BlockSpec((_LB, 128), lambda i: (i, 0))],
    )(o, xf, g1, sc2, sh2, proj_w, proj_b.reshape(1, _D), gw_pad)

    scores = jax.nn.softmax(glog[:, :_E], axis=-1)
    topk_w, topk_i = lax.top_k(scores, _K)
    a_e = topk_i.reshape(-1).astype(jnp.int32)
    a_t = (jnp.arange(_A, dtype=jnp.int32) // _K)
    order = jnp.argsort(a_e)
    s_e = a_e[order]
    s_t = a_t[order]
    counts = jnp.bincount(a_e, length=_E)
    starts = jnp.cumsum(counts) - counts
    nb = (counts + _T - 1) // _T
    cnb = jnp.cumsum(nb)
    blockbase = cnb - nb
    bidx = jnp.arange(_MAXB)
    e_b = jnp.minimum(jnp.searchsorted(cnb, bidx, side='right'),
                      _E - 1).astype(jnp.int32)
    within = bidx - blockbase[e_b]
    start_b = starts[e_b] + within * _T
    len_b = jnp.clip(counts[e_b] - within * _T, 0, _T)
    valid_b = (len_b > 0).astype(jnp.int32)
    t_in = jnp.arange(_T)
    pos = jnp.clip(start_b[:, None] + t_in[None, :], 0, _A - 1)
    inr = t_in[None, :] < len_b[:, None]
    ids = jnp.where(inr, s_t[pos], 0).reshape(-1)

    xs = jnp.take(xm2, ids, axis=0)

    grid_spec = pltpu.PrefetchScalarGridSpec(
        num_scalar_prefetch=2,
        grid=(_NF, _MAXB),
        in_specs=[
            pl.BlockSpec((_T, _D), lambda f, b, be, bv: (b, 0)),
            pl.BlockSpec((1, _FC, _D), lambda f, b, be, bv: (be[b], f, 0)),
            pl.BlockSpec((1, _FC, _D), lambda f, b, be, bv: (be[b], f, 0)),
            pl.BlockSpec((1, _D, _FC), lambda f, b, be, bv: (be[b], 0, f)),
        ],
        out_specs=pl.BlockSpec((_T, _D), lambda f, b, be, bv: (b, 0)),
        scratch_shapes=[pltpu.VMEM((_MAXB * _T, _D), f32)],
    )
    outs = pl.pallas_call(
        _moe_body,
        grid_spec=grid_spec,
        out_shape=jax.ShapeDtypeStruct((_MAXB * _T, _D), f32),
    )(e_b, valid_b, xs, eg_w, eu_w, ed_w)

    invpos = jnp.argsort(order)
    ofs = jnp.arange(_A) - starts[s_e]
    padrow_s = (blockbase[s_e] + ofs // _T) * _T + ofs % _T
    rowof = padrow_s[invpos].reshape(_L, _K)
    y = (topk_w[:, 0:1] * jnp.take(outs, rowof[:, 0], axis=0)
         + topk_w[:, 1:2] * jnp.take(outs, rowof[:, 1], axis=0))

    out = pl.pallas_call(
        _shared_body,
        grid=(nl,),
        out_shape=jax.ShapeDtypeStruct((_L, _D), f32),
        in_specs=[row_spec, row_spec, row_spec, one_spec,
                  pl.BlockSpec((_FS, _D), lambda i: (0, 0)),
                  pl.BlockSpec((_FS, _D), lambda i: (0, 0)),
                  pl.BlockSpec((_D, _FS), lambda i: (0, 0))],
        out_specs=row_spec,
    )(xm2, x1, y, g2, sg_w, su_w, sd_w)

    return out.reshape(_B, _L, _D)


# speedup vs baseline: 1.6358x; 1.2149x over previous
import jax
import jax.numpy as jnp
import numpy as np
from jax import lax
from jax.experimental import pallas as pl
from jax.experimental.pallas import tpu as pltpu

_B, _L, _D, _H = 1, 2048, 1024, 16
_Dh = _D // _H
_E, _K = 8, 2
_F = 4 * _D
_FS = 2 * _D

_LB = 256
_T = 256
_A = _L * _K
_MAXB = _A // _T + _E
_FC = 1024
_NF = _F // _FC


def _mm_nt(a, b):
    return lax.dot_general(a, b, (((1,), (1,)), ((), ())),
                           preferred_element_type=jnp.float32)


def _mm_nn(a, b):
    return lax.dot_general(a, b, (((1,), (0,)), ((), ())),
                           preferred_element_type=jnp.float32)


def _layernorm(x):
    mu = jnp.mean(x, axis=-1, keepdims=True)
    var = jnp.mean((x - mu) ** 2, axis=-1, keepdims=True)
    return (x - mu) * lax.rsqrt(var + 1e-6)


def _mod_body(c_ref, w_ref, b_ref, o_ref):
    cs = jax.nn.silu(c_ref[...])
    o_ref[...] = _mm_nt(cs, w_ref[...]) + b_ref[...]


def _qkv_body(x_ref, s1_ref, sh1_ref, w_ref, b_ref, qs_ref, ks_ref,
              q_out, k_out, v_out):
    x = x_ref[...]
    xm = (1.0 + s1_ref[...]) * _layernorm(x) + sh1_ref[...]
    qkv = _mm_nt(xm, w_ref[...]) + b_ref[...]
    q = qkv[:, :_D]
    k = qkv[:, _D:2 * _D]
    v = qkv[:, 2 * _D:]
    r = lax.broadcasted_iota(jnp.int32, (_D, _H), 0) // _Dh
    c = lax.broadcasted_iota(jnp.int32, (_D, _H), 1)
    m = (r == c).astype(jnp.float32)

    def rms(t, scale_full):
        ssq = _mm_nn(t * t, m)
        rr = lax.rsqrt(ssq / _Dh + 1e-6)
        rrf = _mm_nt(rr, m)
        return t * rrf * scale_full

    q_out[...] = rms(q, qs_ref[...])
    k_out[...] = rms(k, ks_ref[...])
    v_out[...] = v


def _attn_body(q_ref, k_ref, v_ref, o_ref):
    for i in range(2):
        sl = slice(i * _Dh, (i + 1) * _Dh)
        q = q_ref[:, sl]
        k = k_ref[:, sl]
        v = v_ref[:, sl]
        logits = _mm_nt(q, k) * (1.0 / np.sqrt(_Dh))
        p = jnp.exp(logits)
        o_ref[:, sl] = _mm_nn(p, v) / jnp.sum(p, axis=-1, keepdims=True)


def _proj_body(o_ref, x_ref, g1_ref, s2_ref, sh2_ref, w_ref, b_ref, gw_ref,
               x1_out, xm2_out, gl_out):
    p = _mm_nt(o_ref[...], w_ref[...]) + b_ref[...]
    x1 = x_ref[...] + g1_ref[...] * p
    xm2 = (1.0 + s2_ref[...]) * _layernorm(x1) + sh2_ref[...]
    x1_out[...] = x1
    xm2_out[...] = xm2
    gl_out[...] = _mm_nt(xm2, gw_ref[...])


def _moe_body(be_ref, bv_ref, xs_ref, eg_ref, eu_ref, ed_ref, out_ref,
              acc_ref):
    f = pl.program_id(0)
    b = pl.program_id(1)
    sl = pl.ds(b * _T, _T)

    @pl.when(f == 0)
    def _():
        acc_ref[sl, :] = jnp.zeros((_T, _D), jnp.float32)

    @pl.when(bv_ref[b] > 0)
    def _():
        xs = xs_ref[...]
        g = _mm_nt(xs, eg_ref[0])
        u = _mm_nt(xs, eu_ref[0])
        h = jax.nn.silu(g) * u
        acc_ref[sl, :] += lax.dot_general(
            h, ed_ref[0], (((1,), (1,)), ((), ())),
            preferred_element_type=jnp.float32)

    out_ref[...] = acc_ref[sl, :]


def _shared_body(xm2_ref, x1_ref, y_ref, g2_ref, sg_ref, su_ref, sd_ref,
                 out_ref):
    t = xm2_ref[...]
    g = _mm_nt(t, sg_ref[...])
    u = _mm_nt(t, su_ref[...])
    h = jax.nn.silu(g) * u
    sh = _mm_nt(h, sd_ref[...])
    out_ref[...] = x1_ref[...] + g2_ref[...] * (y_ref[...] + sh)


def _full(shape):
    return pl.BlockSpec(shape, lambda *_: tuple(0 for _ in shape))


def kernel(x, c, mod_w, mod_b, qkv_w, qkv_b, q_scale, k_scale, proj_w,
           proj_b, gate_w, eg_w, eu_w, ed_w, sg_w, su_w, sd_w):
    f32 = jnp.float32
    xf = x.reshape(_L, _D)
    nl = _L // _LB

    vec = pl.pallas_call(
        _mod_body,
        out_shape=jax.ShapeDtypeStruct((1, 6 * _D), f32),
        in_specs=[_full((1, _D)), _full((6 * _D, _D)), _full((1, 6 * _D))],
        out_specs=_full((1, 6 * _D)),
    )(c.reshape(1, _D), mod_w, mod_b.reshape(1, 6 * _D))
    sh1 = vec[:, 0:_D]
    sc1 = vec[:, _D:2 * _D]
    g1 = vec[:, 2 * _D:3 * _D]
    sh2 = vec[:, 3 * _D:4 * _D]
    sc2 = vec[:, 4 * _D:5 * _D]
    g2 = vec[:, 5 * _D:6 * _D]

    qs_full = jnp.tile(q_scale, _H).reshape(1, _D)
    ks_full = jnp.tile(k_scale, _H).reshape(1, _D)

    row_spec = pl.BlockSpec((_LB, _D), lambda i: (i, 0))
    one_spec = pl.BlockSpec((1, _D), lambda i: (0, 0))
    q, k, v = pl.pallas_call(
        _qkv_body,
        grid=(nl,),
        out_shape=[jax.ShapeDtypeStruct((_L, _D), f32)] * 3,
        in_specs=[row_spec, one_spec, one_spec,
                  pl.BlockSpec((3 * _D, _D), lambda i: (0, 0)),
                  pl.BlockSpec((1, 3 * _D), lambda i: (0, 0)),
                  one_spec, one_spec],
        out_specs=[row_spec] * 3,
    )(xf, sc1, sh1, qkv_w, qkv_b.reshape(1, 3 * _D), qs_full, ks_full)

    pair_spec = pl.BlockSpec((_L, 2 * _Dh), lambda h: (0, h))
    o = pl.pallas_call(
        _attn_body,
        grid=(_H // 2,),
        out_shape=jax.ShapeDtypeStruct((_L, _D), f32),
        in_specs=[pair_spec] * 3,
        out_specs=pair_spec,
    )(q, k, v)

    gw_pad = jnp.zeros((128, _D), f32).at[:_E].set(gate_w)
    x1, xm2, glog = pl.pallas_call(
        _proj_body,
        grid=(nl,),
        out_shape=[jax.ShapeDtypeStruct((_L, _D), f32),
                   jax.ShapeDtypeStruct((_L, _D), f32),
                   jax.ShapeDtypeStruct((_L, 128), f32)],
        in_specs=[row_spec, row_spec, one_spec, one_spec, one_spec,
                  pl.BlockSpec((_D, _D), lambda i: (0, 0)),
                  one_spec,
                  pl.BlockSpec((128, _D), lambda i: (0, 0))],
        out_specs=[row_spec, row_spec,
                   pl.BlockSpec((_LB, 128), lambda i: (i, 0))],
    )(o, xf, g1, sc2, sh2, proj_w, proj_b.reshape(1, _D), gw_pad)

    scores = jax.nn.softmax(glog[:, :_E], axis=-1)
    topk_w, topk_i = lax.top_k(scores, _K)
    a_e = topk_i.reshape(-1).astype(jnp.int32)
    a_t = (jnp.arange(_A, dtype=jnp.int32) // _K)
    order = jnp.argsort(a_e)
    s_e = a_e[order]
    s_t = a_t[order]
    counts = jnp.bincount(a_e, length=_E)
    starts = jnp.cumsum(counts) - counts
    nb = (counts + _T - 1) // _T
    cnb = jnp.cumsum(nb)
    blockbase = cnb - nb
    bidx = jnp.arange(_MAXB)
    e_b = jnp.minimum(jnp.searchsorted(cnb, bidx, side='right'),
                      _E - 1).astype(jnp.int32)
    within = bidx - blockbase[e_b]
    start_b = starts[e_b] + within * _T
    len_b = jnp.clip(counts[e_b] - within * _T, 0, _T)
    valid_b = (len_b > 0).astype(jnp.int32)
    t_in = jnp.arange(_T)
    pos = jnp.clip(start_b[:, None] + t_in[None, :], 0, _A - 1)
    inr = t_in[None, :] < len_b[:, None]
    ids = jnp.where(inr, s_t[pos], 0).reshape(-1)

    xs = jnp.take(xm2, ids, axis=0)

    grid_spec = pltpu.PrefetchScalarGridSpec(
        num_scalar_prefetch=2,
        grid=(_NF, _MAXB),
        in_specs=[
            pl.BlockSpec((_T, _D), lambda f, b, be, bv: (b, 0)),
            pl.BlockSpec((1, _FC, _D), lambda f, b, be, bv: (be[b], f, 0)),
            pl.BlockSpec((1, _FC, _D), lambda f, b, be, bv: (be[b], f, 0)),
            pl.BlockSpec((1, _D, _FC), lambda f, b, be, bv: (be[b], 0, f)),
        ],
        out_specs=pl.BlockSpec((_T, _D), lambda f, b, be, bv: (b, 0)),
        scratch_shapes=[pltpu.VMEM((_MAXB * _T, _D), f32)],
    )
    outs = pl.pallas_call(
        _moe_body,
        grid_spec=grid_spec,
        out_shape=jax.ShapeDtypeStruct((_MAXB * _T, _D), f32),
    )(e_b, valid_b, xs, eg_w, eu_w, ed_w)

    invpos = jnp.argsort(order)
    ofs = jnp.arange(_A) - starts[s_e]
    padrow_s = (blockbase[s_e] + ofs // _T) * _T + ofs % _T
    rowof = padrow_s[invpos].reshape(_L, _K)
    y = (topk_w[:, 0:1] * jnp.take(outs, rowof[:, 0], axis=0)
         + topk_w[:, 1:2] * jnp.take(outs, rowof[:, 1], axis=0))

    out = pl.pallas_call(
        _shared_body,
        grid=(nl,),
        out_shape=jax.ShapeDtypeStruct((_L, _D), f32),
        in_specs=[row_spec, row_spec, row_spec, one_spec,
                  pl.BlockSpec((_FS, _D), lambda i: (0, 0)),
                  pl.BlockSpec((_FS, _D), lambda i: (0, 0)),
                  pl.BlockSpec((_D, _FS), lambda i: (0, 0))],
        out_specs=row_spec,
    )(xm2, x1, y, g2, sg_w, su_w, sd_w)

    return out.reshape(_B, _L, _D)
